# baseline (speedup 1.0000x reference)
import functools

import jax
import jax.numpy as jnp
from jax import lax
from jax.experimental import pallas as pl
from jax.experimental.pallas import tpu as pltpu
from jax.experimental.pallas import tpu_sc as plsc

N = 10000
E = 320000
D = 128
NDEPTH = 2
NPASS = 4
NIN = 5

NC = 2
NS = 16
NW = NC * NS
EPW = E // NW
CH = 80
NCHUNK = EPW // CH
NPAD = 10240

_sc_mesh = plsc.VectorSubcoreMesh(core_axis_name="c", subcore_axis_name="s")


@functools.partial(
    pl.kernel,
    out_type=jax.ShapeDtypeStruct((E, D), jnp.float32),
    mesh=_sc_mesh,
    scratch_types=[
        pltpu.VMEM((NCHUNK, CH), jnp.int32),
        pltpu.VMEM((NCHUNK, CH), jnp.int32),
        pltpu.VMEM((2, CH, D), jnp.float32),
        pltpu.VMEM((2, CH, D), jnp.float32),
        pltpu.VMEM((2, CH, D), jnp.float32),
        pltpu.SemaphoreType.DMA,
        pltpu.SemaphoreType.DMA,
        pltpu.SemaphoreType.DMA,
        pltpu.SemaphoreType.DMA,
    ],
)
def _sc_gather_diff(vv, sidx_h, didx_h, out, sidx, didx, sbuf, dbuf, obuf,
                    ssem, dsem, osem, isem):
    wid = lax.axis_index("s") * NC + lax.axis_index("c")
    base = wid * EPW
    def iload(j, _):
        pltpu.async_copy(sidx_h.at[pl.ds(base + j * CH, CH)], sidx.at[j], isem)
        pltpu.async_copy(didx_h.at[pl.ds(base + j * CH, CH)], didx.at[j], isem)
        return 0

    lax.fori_loop(0, NCHUNK, iload, 0)

    def idrain(j, _):
        pltpu.make_async_copy(
            sidx_h.at[pl.ds(base + j * CH, CH)], sidx.at[j], isem).wait()
        pltpu.make_async_copy(
            didx_h.at[pl.ds(base + j * CH, CH)], didx.at[j], isem).wait()
        return 0

    lax.fori_loop(0, NCHUNK, idrain, 0)
    pltpu.async_copy(vv.at[sidx.at[0]], sbuf.at[0], ssem)
    pltpu.async_copy(vv.at[didx.at[0]], dbuf.at[0], dsem)

    def body(j, _):
        slot = lax.rem(j, 2)
        nslot = lax.rem(j + 1, 2)

        @pl.when(j + 1 < NCHUNK)
        def _():
            pltpu.async_copy(vv.at[sidx.at[j + 1]], sbuf.at[nslot], ssem)
            pltpu.async_copy(vv.at[didx.at[j + 1]], dbuf.at[nslot], dsem)

        pltpu.make_async_copy(vv.at[sidx.at[j]], sbuf.at[slot], ssem).wait()
        pltpu.make_async_copy(vv.at[didx.at[j]], dbuf.at[slot], dsem).wait()

        @pl.when(j >= 2)
        def _():
            pltpu.make_async_copy(
                obuf.at[slot], out.at[pl.ds(base + (j - 2) * CH, CH)], osem
            ).wait()

        def crow(r, _):
            for c in range(8):
                sl = pl.ds(c * 16, 16)
                obuf[slot, r, sl] = sbuf[slot, r, sl] - dbuf[slot, r, sl]
            return 0

        lax.fori_loop(0, CH, crow, 0)
        pltpu.async_copy(obuf.at[slot], out.at[pl.ds(base + j * CH, CH)], osem)
        return 0

    lax.fori_loop(0, NCHUNK, body, 0)
    for jj in (NCHUNK - 2, NCHUNK - 1):
        pltpu.make_async_copy(
            obuf.at[jj % 2], out.at[pl.ds(base + jj * CH, CH)], osem
        ).wait()


HALF = NPAD // 2
ACCR = HALF + 128
EPS = E // NS
NCHUNK2 = EPS // CH
RPS = HALF // NS


@functools.partial(
    pl.kernel,
    out_type=jax.ShapeDtypeStruct((NC, HALF, D), jnp.float32),
    mesh=_sc_mesh,
    scratch_types=[
        pltpu.VMEM((NCHUNK2, CH), jnp.int32),
        pltpu.VMEM((2, CH, D), jnp.float32),
        pltpu.VMEM((RPS // 2, D), jnp.float32),
        pltpu.VMEM_SHARED((ACCR, D), jnp.float32),
        pltpu.SemaphoreType.DMA,
        pltpu.SemaphoreType.DMA,
    ],
)
def _sc_scatter_sum(e_h, didx_h, out, didx, ebuf, zbuf, acc, esem, isem):
    cid = lax.axis_index("c")
    sid = lax.axis_index("s")
    base = sid * EPS

    def iload(j, _):
        pltpu.async_copy(didx_h.at[pl.ds(base + j * CH, CH)], didx.at[j], isem)
        return 0

    lax.fori_loop(0, NCHUNK2, iload, 0)

    def zrow(r, _):
        for c in range(8):
            zbuf[r, pl.ds(c * 16, 16)] = jnp.zeros((16,), jnp.float32)
        return 0

    lax.fori_loop(0, RPS // 2, zrow, 0)
    for t in range(2):
        pltpu.sync_copy(zbuf, acc.at[pl.ds(sid * RPS + t * (RPS // 2), RPS // 2)])

    def idrain(j, _):
        pltpu.make_async_copy(
            didx_h.at[pl.ds(base + j * CH, CH)], didx.at[j], isem).wait()
        return 0

    lax.fori_loop(0, NCHUNK2, idrain, 0)

    off = cid * HALF

    def rrow(r, _):
        for c in range(CH // 16):
            sl = pl.ds(c * 16, 16)
            vi = didx[r, sl]
            u = vi - off
            ok = (u >= 0) & (u < HALF)
            didx[r, sl] = jnp.where(ok, u, HALF + (vi & 127))
        return 0

    lax.fori_loop(0, NCHUNK2, rrow, 0)
    plsc.subcore_barrier()
    pltpu.async_copy(e_h.at[pl.ds(base, CH)], ebuf.at[0], esem)

    def body(j, _):
        slot = lax.rem(j, 2)
        nslot = lax.rem(j + 1, 2)

        @pl.when(j + 1 < NCHUNK2)
        def _():
            pltpu.async_copy(
                e_h.at[pl.ds(base + (j + 1) * CH, CH)], ebuf.at[nslot], esem
            )

        pltpu.make_async_copy(
            e_h.at[pl.ds(base + j * CH, CH)], ebuf.at[slot], esem
        ).wait()
        pltpu.sync_copy(ebuf.at[slot], acc.at[didx.at[j]], add=True)
        return 0

    lax.fori_loop(0, NCHUNK2, body, 0)
    plsc.subcore_barrier()
    pltpu.sync_copy(
        acc.at[pl.ds(sid * RPS, RPS)], out.at[cid, pl.ds(sid * RPS, RPS)]
    )


def _bdot(a, b):
    return jnp.dot(a.astype(jnp.bfloat16), b.astype(jnp.bfloat16),
                   preferred_element_type=jnp.float32)


def _res_mlp(x, w1, b1, w2, b2, g, bta):
    for i in range(NDEPTH):
        h = jnp.maximum(_bdot(x, w1[i]) + b1[i][None, :], 0.0)
        h = jnp.maximum(_bdot(h, w2[i]) + b2[i][None, :], 0.0)
        mu = jnp.mean(h, axis=-1, keepdims=True)
        d = h - mu
        var = jnp.mean(d * d, axis=-1, keepdims=True)
        h = d * lax.rsqrt(var + 1e-5) * g[i][None, :] + bta[i][None, :]
        x = x + h
    return x


def _pack_cols(xf):
    vb = xf.astype(jnp.bfloat16)
    a = lax.bitcast_convert_type(vb[:, :DP], jnp.uint16).astype(jnp.uint32)
    b = lax.bitcast_convert_type(vb[:, DP:], jnp.uint16).astype(jnp.uint32)
    return a | (b << 16)


def _unpack_cols(w):
    lo = lax.bitcast_convert_type((w & 0xFFFF).astype(jnp.uint16), jnp.bfloat16)
    hi = lax.bitcast_convert_type((w >> 16).astype(jnp.uint16), jnp.bfloat16)
    return jnp.concatenate([lo, hi], axis=1).astype(jnp.float32)


def _wspec(shape):
    return pl.BlockSpec(shape, lambda i: tuple(0 for _ in shape))


_W_SPECS = [
    _wspec((NDEPTH, D, D)), _wspec((NDEPTH, D)),
    _wspec((NDEPTH, D, D)), _wspec((NDEPTH, D)),
    _wspec((NDEPTH, D)), _wspec((NDEPTH, D)),
]

BE = 4000
BN = 2000


def _edge_body0(x_ref, w1, b1, w2, b2, g, bta, out_ref):
    out_ref[...] = _res_mlp(x_ref[...], w1, b1, w2, b2, g, bta)


def _edge_body1(x_ref, e_ref, w1, b1, w2, b2, g, bta, out_ref):
    x = x_ref[...] + e_ref[...]
    out_ref[...] = _res_mlp(x, w1, b1, w2, b2, g, bta)


_edge_call0 = pl.pallas_call(
    _edge_body0,
    grid=(E // BE,),
    in_specs=[pl.BlockSpec((BE, D), lambda i: (i, 0))] + _W_SPECS,
    out_specs=pl.BlockSpec((BE, D), lambda i: (i, 0)),
    out_shape=jax.ShapeDtypeStruct((E, D), jnp.float32),
)

_edge_call1 = pl.pallas_call(
    _edge_body1,
    grid=(E // BE,),
    in_specs=[pl.BlockSpec((BE, D), lambda i: (i, 0)),
              pl.BlockSpec((BE, D), lambda i: (i, 0))] + _W_SPECS,
    out_specs=pl.BlockSpec((BE, D), lambda i: (i, 0)),
    out_shape=jax.ShapeDtypeStruct((E, D), jnp.float32),
)


def _node_body(vv_ref, a_ref, w1, b1, w2, b2, g, bta, out_ref):
    x = vv_ref[...] + a_ref[...]
    out_ref[...] = _res_mlp(x, w1, b1, w2, b2, g, bta)


_node_call = pl.pallas_call(
    _node_body,
    grid=(N // BN,),
    in_specs=[pl.BlockSpec((BN, D), lambda i: (i, 0)),
              pl.BlockSpec((BN, D), lambda i: (i, 0))] + _W_SPECS,
    out_specs=pl.BlockSpec((BN, D), lambda i: (i, 0)),
    out_shape=jax.ShapeDtypeStruct((N, D), jnp.float32),
)


def _enc_body(v_ref, w_ref, b_ref, out_ref):
    out_ref[...] = (
        jnp.dot(v_ref[...], w_ref[...], preferred_element_type=jnp.float32)
        + b_ref[...][None, :]
    )


_enc_call = pl.pallas_call(
    _enc_body,
    grid=(N // BN,),
    in_specs=[pl.BlockSpec((BN, 8), lambda i: (i, 0)),
              _wspec((8, D)), _wspec((D,))],
    out_specs=pl.BlockSpec((BN, D), lambda i: (i, 0)),
    out_shape=jax.ShapeDtypeStruct((N, D), jnp.float32),
)

_dec_call = pl.pallas_call(
    _enc_body,
    grid=(N // BN,),
    in_specs=[pl.BlockSpec((BN, D), lambda i: (i, 0)),
              _wspec((D, 8)), _wspec((8,))],
    out_specs=pl.BlockSpec((BN, 8), lambda i: (i, 0)),
    out_shape=jax.ShapeDtypeStruct((N, 8), jnp.float32),
)


def kernel(v, ij, enc_W, enc_b, dec_W, dec_b,
           edge_W1, edge_b1, edge_W2, edge_b2, edge_g, edge_beta,
           node_W1, node_b1, node_W2, node_b2, node_g, node_beta):
    src = ij[0]
    dst = ij[1]
    v8 = jnp.pad(v.reshape(N, NIN), ((0, 0), (0, 8 - NIN)))
    encW8 = jnp.pad(enc_W, ((0, 8 - NIN), (0, 0)))
    decW8 = jnp.pad(dec_W, ((0, 0), (0, 8 - NIN)))
    decb8 = jnp.pad(dec_b, (0, 8 - NIN))

    vv = _enc_call(v8, encW8, enc_b)
    e = None
    for _ in range(NPASS):
        x = _sc_gather_diff(vv, src, dst)
        if e is None:
            e = _edge_call0(x, edge_W1, edge_b1, edge_W2, edge_b2,
                            edge_g, edge_beta)
        else:
            e = _edge_call1(x, e, edge_W1, edge_b1, edge_W2, edge_b2,
                            edge_g, edge_beta)
        acc = _sc_scatter_sum(e, dst)
        eij = acc.reshape(NPAD, D)
        vv = _node_call(vv, eij, node_W1, node_b1, node_W2, node_b2,
                        node_g, node_beta)
    out8 = _dec_call(vv, decW8, decb8)
    return out8[:, :NIN].reshape(1, N, NIN)


# speedup vs baseline: 1.2337x; 1.2337x over previous
import functools

import jax
import jax.numpy as jnp
from jax import lax
from jax.experimental import pallas as pl
from jax.experimental.pallas import tpu as pltpu
from jax.experimental.pallas import tpu_sc as plsc

N = 10000
E = 320000
D = 128
NDEPTH = 2
NPASS = 4
NIN = 5
NSPLIT = 2
E2 = E // NSPLIT

NC = 2
NS = 16
NW = NC * NS
NPAD = 10240
HALF = NPAD // 2
ACCR = HALF + 128
RPS = HALF // NS

_sc_mesh = plsc.VectorSubcoreMesh(core_axis_name="c", subcore_axis_name="s")


def _make_gather(e_n, ch):
    epw = e_n // NW
    nchunk = epw // ch

    @functools.partial(
        pl.kernel,
        out_type=jax.ShapeDtypeStruct((e_n, D), jnp.float32),
        mesh=_sc_mesh,
        scratch_types=[
            pltpu.VMEM((nchunk, ch), jnp.int32),
            pltpu.VMEM((nchunk, ch), jnp.int32),
            pltpu.VMEM((2, ch, D), jnp.float32),
            pltpu.VMEM((2, ch, D), jnp.float32),
            pltpu.VMEM((2, ch, D), jnp.float32),
            pltpu.SemaphoreType.DMA,
            pltpu.SemaphoreType.DMA,
            pltpu.SemaphoreType.DMA,
            pltpu.SemaphoreType.DMA,
        ],
    )
    def gather(vv, sidx_h, didx_h, out, sidx, didx, sbuf, dbuf, obuf,
               ssem, dsem, osem, isem):
        wid = lax.axis_index("s") * NC + lax.axis_index("c")
        base = wid * epw

        def iload(j, _):
            pltpu.async_copy(sidx_h.at[pl.ds(base + j * ch, ch)],
                             sidx.at[j], isem)
            pltpu.async_copy(didx_h.at[pl.ds(base + j * ch, ch)],
                             didx.at[j], isem)
            return 0

        lax.fori_loop(0, nchunk, iload, 0)

        def idrain(j, _):
            pltpu.make_async_copy(
                sidx_h.at[pl.ds(base + j * ch, ch)], sidx.at[j], isem).wait()
            pltpu.make_async_copy(
                didx_h.at[pl.ds(base + j * ch, ch)], didx.at[j], isem).wait()
            return 0

        lax.fori_loop(0, nchunk, idrain, 0)
        pltpu.async_copy(vv.at[sidx.at[0]], sbuf.at[0], ssem)
        pltpu.async_copy(vv.at[didx.at[0]], dbuf.at[0], dsem)

        def body(j, _):
            slot = lax.rem(j, 2)
            nslot = lax.rem(j + 1, 2)

            @pl.when(j + 1 < nchunk)
            def _():
                pltpu.async_copy(vv.at[sidx.at[j + 1]], sbuf.at[nslot], ssem)
                pltpu.async_copy(vv.at[didx.at[j + 1]], dbuf.at[nslot], dsem)

            pltpu.make_async_copy(vv.at[sidx.at[j]], sbuf.at[slot], ssem).wait()
            pltpu.make_async_copy(vv.at[didx.at[j]], dbuf.at[slot], dsem).wait()

            @pl.when(j >= 2)
            def _():
                pltpu.make_async_copy(
                    obuf.at[slot], out.at[pl.ds(base + (j - 2) * ch, ch)], osem
                ).wait()

            def crow(r, _):
                for c in range(D // 16):
                    sl = pl.ds(c * 16, 16)
                    obuf[slot, r, sl] = sbuf[slot, r, sl] - dbuf[slot, r, sl]
                return 0

            lax.fori_loop(0, ch, crow, 0)
            pltpu.async_copy(obuf.at[slot],
                             out.at[pl.ds(base + j * ch, ch)], osem)
            return 0

        lax.fori_loop(0, nchunk, body, 0)
        for jj in (nchunk - 2, nchunk - 1):
            pltpu.make_async_copy(
                obuf.at[jj % 2], out.at[pl.ds(base + jj * ch, ch)], osem
            ).wait()

    return gather


def _make_scatter(e_n, ch):
    eps = e_n // NS
    nchunk = eps // ch

    @functools.partial(
        pl.kernel,
        out_type=jax.ShapeDtypeStruct((NC, HALF, D), jnp.float32),
        mesh=_sc_mesh,
        scratch_types=[
            pltpu.VMEM((nchunk, ch), jnp.int32),
            pltpu.VMEM((2, ch, D), jnp.float32),
            pltpu.VMEM((RPS // 2, D), jnp.float32),
            pltpu.VMEM_SHARED((ACCR, D), jnp.float32),
            pltpu.SemaphoreType.DMA,
            pltpu.SemaphoreType.DMA,
        ],
    )
    def scatter(e_h, didx_h, out, didx, ebuf, zbuf, acc, esem, isem):
        cid = lax.axis_index("c")
        sid = lax.axis_index("s")
        base = sid * eps

        def iload(j, _):
            pltpu.async_copy(didx_h.at[pl.ds(base + j * ch, ch)],
                             didx.at[j], isem)
            return 0

        lax.fori_loop(0, nchunk, iload, 0)

        def zrow(r, _):
            for c in range(D // 16):
                zbuf[r, pl.ds(c * 16, 16)] = jnp.zeros((16,), jnp.float32)
            return 0

        lax.fori_loop(0, RPS // 2, zrow, 0)
        for t in range(2):
            pltpu.sync_copy(
                zbuf, acc.at[pl.ds(sid * RPS + t * (RPS // 2), RPS // 2)])

        def idrain(j, _):
            pltpu.make_async_copy(
                didx_h.at[pl.ds(base + j * ch, ch)], didx.at[j], isem).wait()
            return 0

        lax.fori_loop(0, nchunk, idrain, 0)

        off = cid * HALF

        def rrow(r, _):
            for c in range(ch // 16):
                sl = pl.ds(c * 16, 16)
                vi = didx[r, sl]
                u = vi - off
                ok = (u >= 0) & (u < HALF)
                didx[r, sl] = jnp.where(ok, u, HALF + (vi & 127))
            return 0

        lax.fori_loop(0, nchunk, rrow, 0)
        plsc.subcore_barrier()
        pltpu.async_copy(e_h.at[pl.ds(base, ch)], ebuf.at[0], esem)

        def body(j, _):
            slot = lax.rem(j, 2)
            nslot = lax.rem(j + 1, 2)

            @pl.when(j + 1 < nchunk)
            def _():
                pltpu.async_copy(
                    e_h.at[pl.ds(base + (j + 1) * ch, ch)], ebuf.at[nslot],
                    esem)

            pltpu.make_async_copy(
                e_h.at[pl.ds(base + j * ch, ch)], ebuf.at[slot], esem).wait()
            pltpu.sync_copy(ebuf.at[slot], acc.at[didx.at[j]], add=True)
            return 0

        lax.fori_loop(0, nchunk, body, 0)
        plsc.subcore_barrier()
        pltpu.sync_copy(
            acc.at[pl.ds(sid * RPS, RPS)], out.at[cid, pl.ds(sid * RPS, RPS)]
        )

    return scatter


_gather_c = _make_gather(E2, 40)
_scatter_c = _make_scatter(E2, 80)


def _bdot(a, b):
    return jnp.dot(a.astype(jnp.bfloat16), b.astype(jnp.bfloat16),
                   preferred_element_type=jnp.float32)


def _res_mlp(x, w1, b1, w2, b2, g, bta):
    for i in range(NDEPTH):
        h = jnp.maximum(_bdot(x, w1[i]) + b1[i][None, :], 0.0)
        h = jnp.maximum(_bdot(h, w2[i]) + b2[i][None, :], 0.0)
        mu = jnp.mean(h, axis=-1, keepdims=True)
        d = h - mu
        var = jnp.mean(d * d, axis=-1, keepdims=True)
        h = d * lax.rsqrt(var + 1e-5) * g[i][None, :] + bta[i][None, :]
        x = x + h
    return x


def _wspec(shape):
    return pl.BlockSpec(shape, lambda i: tuple(0 for _ in shape))


_W_SPECS = [
    _wspec((NDEPTH, D, D)), _wspec((NDEPTH, D)),
    _wspec((NDEPTH, D, D)), _wspec((NDEPTH, D)),
    _wspec((NDEPTH, D)), _wspec((NDEPTH, D)),
]

BE = 4000
BN = 2000


def _edge_body0(x_ref, w1, b1, w2, b2, g, bta, out_ref):
    out_ref[...] = _res_mlp(x_ref[...], w1, b1, w2, b2, g, bta)


def _edge_body1(x_ref, e_ref, w1, b1, w2, b2, g, bta, out_ref):
    x = x_ref[...] + e_ref[...]
    out_ref[...] = _res_mlp(x, w1, b1, w2, b2, g, bta)


def _make_edge(e_n):
    call0 = pl.pallas_call(
        _edge_body0,
        grid=(e_n // BE,),
        in_specs=[pl.BlockSpec((BE, D), lambda i: (i, 0))] + _W_SPECS,
        out_specs=pl.BlockSpec((BE, D), lambda i: (i, 0)),
        out_shape=jax.ShapeDtypeStruct((e_n, D), jnp.float32),
    )
    call1 = pl.pallas_call(
        _edge_body1,
        grid=(e_n // BE,),
        in_specs=[pl.BlockSpec((BE, D), lambda i: (i, 0)),
                  pl.BlockSpec((BE, D), lambda i: (i, 0))] + _W_SPECS,
        out_specs=pl.BlockSpec((BE, D), lambda i: (i, 0)),
        out_shape=jax.ShapeDtypeStruct((e_n, D), jnp.float32),
    )
    return call0, call1


_edge_call0, _edge_call1 = _make_edge(E2)


def _node_body(vv_ref, a0_ref, a1_ref, w1, b1, w2, b2, g, bta, out_ref):
    x = vv_ref[...] + a0_ref[...] + a1_ref[...]
    out_ref[...] = _res_mlp(x, w1, b1, w2, b2, g, bta)


_node_call = pl.pallas_call(
    _node_body,
    grid=(N // BN,),
    in_specs=[pl.BlockSpec((BN, D), lambda i: (i, 0)),
              pl.BlockSpec((BN, D), lambda i: (i, 0)),
              pl.BlockSpec((BN, D), lambda i: (i, 0))] + _W_SPECS,
    out_specs=pl.BlockSpec((BN, D), lambda i: (i, 0)),
    out_shape=jax.ShapeDtypeStruct((N, D), jnp.float32),
)


def _enc_body(v_ref, w_ref, b_ref, out_ref):
    out_ref[...] = (
        jnp.dot(v_ref[...], w_ref[...], preferred_element_type=jnp.float32)
        + b_ref[...][None, :]
    )


_enc_call = pl.pallas_call(
    _enc_body,
    grid=(N // BN,),
    in_specs=[pl.BlockSpec((BN, 8), lambda i: (i, 0)),
              _wspec((8, D)), _wspec((D,))],
    out_specs=pl.BlockSpec((BN, D), lambda i: (i, 0)),
    out_shape=jax.ShapeDtypeStruct((N, D), jnp.float32),
)

_dec_call = pl.pallas_call(
    _enc_body,
    grid=(N // BN,),
    in_specs=[pl.BlockSpec((BN, D), lambda i: (i, 0)),
              _wspec((D, 8)), _wspec((8,))],
    out_specs=pl.BlockSpec((BN, 8), lambda i: (i, 0)),
    out_shape=jax.ShapeDtypeStruct((N, 8), jnp.float32),
)


def kernel(v, ij, enc_W, enc_b, dec_W, dec_b,
           edge_W1, edge_b1, edge_W2, edge_b2, edge_g, edge_beta,
           node_W1, node_b1, node_W2, node_b2, node_g, node_beta):
    srcs = [ij[0, c * E2:(c + 1) * E2] for c in range(NSPLIT)]
    dsts = [ij[1, c * E2:(c + 1) * E2] for c in range(NSPLIT)]
    v8 = jnp.pad(v.reshape(N, NIN), ((0, 0), (0, 8 - NIN)))
    encW8 = jnp.pad(enc_W, ((0, 8 - NIN), (0, 0)))
    decW8 = jnp.pad(dec_W, ((0, 0), (0, 8 - NIN)))
    decb8 = jnp.pad(dec_b, (0, 8 - NIN))
    ew = (edge_W1, edge_b1, edge_W2, edge_b2, edge_g, edge_beta)
    nw = (node_W1, node_b1, node_W2, node_b2, node_g, node_beta)

    vv = _enc_call(v8, encW8, enc_b)
    es = [None] * NSPLIT
    for _ in range(NPASS):
        xs = [_gather_c(vv, srcs[c], dsts[c]) for c in range(NSPLIT)]
        es = [
            _edge_call0(xs[c], *ew) if es[c] is None
            else _edge_call1(xs[c], es[c], *ew)
            for c in range(NSPLIT)
        ]
        accs = [_scatter_c(es[c], dsts[c]) for c in range(NSPLIT)]
        vv = _node_call(vv, accs[0].reshape(NPAD, D), accs[1].reshape(NPAD, D),
                        *nw)
    out8 = _dec_call(vv, decW8, decb8)
    return out8[:, :NIN].reshape(1, N, NIN)
